# initial kernel scaffold (unmeasured)
import jax
import jax.numpy as jnp
from jax import lax
from jax.experimental import pallas as pl
from jax.experimental.pallas import tpu as pltpu


def kernel(x, pi):
    def body(pi_ref, x_ref, out_ref, send_sem, recv_sem):
        my_x = lax.axis_index("x")
        my_y = lax.axis_index("y")
        my_z = lax.axis_index("z")

        dst_z = pi_ref[my_z]

        rdma = pltpu.make_async_remote_copy(
            src_ref=x_ref,
            dst_ref=out_ref,
            send_sem=send_sem,
            recv_sem=recv_sem,
            device_id=(my_x, my_y, dst_z),
            device_id_type=pl.DeviceIdType.MESH,
        )
        rdma.start()
        rdma.wait_send()
        rdma.wait_recv()

    out_shape = jax.ShapeDtypeStruct(x.shape, jnp.float32)
    return pl.pallas_call(
        body,
        out_shape=out_shape,
        in_specs=[
            pl.BlockSpec(memory_space=pltpu.SMEM),
            pl.BlockSpec(memory_space=pltpu.ANY),
        ],
        out_specs=pl.BlockSpec(memory_space=pltpu.ANY),
        scratch_shapes=[
            pltpu.SemaphoreType.DMA,
            pltpu.SemaphoreType.DMA,
        ],
    )(pi, x)


# baseline (device time: 396075 ns/iter reference)
import jax
import jax.numpy as jnp
from jax import lax
from jax.experimental import pallas as pl
from jax.experimental.pallas import tpu as pltpu


def kernel(x, pi):
    def body(pi_ref, x_ref, out_ref, send_sem, recv_sem):
        my_x = lax.axis_index("x")
        my_y = lax.axis_index("y")
        my_z = lax.axis_index("z")

        dst_z = pi_ref[my_z]

        rdma = pltpu.make_async_remote_copy(
            src_ref=x_ref,
            dst_ref=out_ref,
            send_sem=send_sem,
            recv_sem=recv_sem,
            device_id=(my_x, my_y, dst_z),
            device_id_type=pl.DeviceIdType.MESH,
        )
        rdma.start()
        rdma.wait_send()
        rdma.wait_recv()

    out_shape = jax.ShapeDtypeStruct(x.shape, jnp.float32)
    return pl.pallas_call(
        body,
        out_shape=out_shape,
        in_specs=[
            pl.BlockSpec(memory_space=pltpu.MemorySpace.SMEM),
            pl.BlockSpec(memory_space=pl.ANY),
        ],
        out_specs=pl.BlockSpec(memory_space=pl.ANY),
        scratch_shapes=[
            pltpu.SemaphoreType.DMA,
            pltpu.SemaphoreType.DMA,
        ],
    )(pi, x)


# device time: 231356 ns/iter; 1.7120x vs baseline; 1.7120x over previous
import jax
import jax.numpy as jnp
from jax import lax
from jax.experimental import pallas as pl
from jax.experimental.pallas import tpu as pltpu

N_CHUNKS = 16


def kernel(x, pi):
    _, m, n = x.shape
    half = m // 2
    ch = half // N_CHUNKS

    def body(pi_ref, x_ref, out_ref, zs_send, zs_recv, xs_send, xs_recv):
        my_x = lax.axis_index("x")
        my_y = lax.axis_index("y")
        my_z = lax.axis_index("z")

        dst_z = pi_ref[my_z]
        nbr_x = 1 - my_x
        base = my_x * half

        z_rdmas = []
        for c in range(N_CHUNKS):
            off = base + c * ch
            r = pltpu.make_async_remote_copy(
                src_ref=x_ref.at[0, pl.ds(off, ch)],
                dst_ref=out_ref.at[0, pl.ds(off, ch)],
                send_sem=zs_send.at[c],
                recv_sem=zs_recv.at[c],
                device_id=(my_x, my_y, dst_z),
                device_id_type=pl.DeviceIdType.MESH,
            )
            r.start()
            z_rdmas.append(r)

        x_rdmas = []
        for c in range(N_CHUNKS):
            z_rdmas[c].wait_recv()
            off = base + c * ch
            r = pltpu.make_async_remote_copy(
                src_ref=out_ref.at[0, pl.ds(off, ch)],
                dst_ref=out_ref.at[0, pl.ds(off, ch)],
                send_sem=xs_send.at[c],
                recv_sem=xs_recv.at[c],
                device_id=(nbr_x, my_y, my_z),
                device_id_type=pl.DeviceIdType.MESH,
            )
            r.start()
            x_rdmas.append(r)

        for c in range(N_CHUNKS):
            x_rdmas[c].wait_recv()
        for c in range(N_CHUNKS):
            z_rdmas[c].wait_send()
            x_rdmas[c].wait_send()

    out_shape = jax.ShapeDtypeStruct(x.shape, jnp.float32)
    return pl.pallas_call(
        body,
        out_shape=out_shape,
        in_specs=[
            pl.BlockSpec(memory_space=pltpu.MemorySpace.SMEM),
            pl.BlockSpec(memory_space=pl.ANY),
        ],
        out_specs=pl.BlockSpec(memory_space=pl.ANY),
        scratch_shapes=[
            pltpu.SemaphoreType.DMA((N_CHUNKS,)),
            pltpu.SemaphoreType.DMA((N_CHUNKS,)),
            pltpu.SemaphoreType.DMA((N_CHUNKS,)),
            pltpu.SemaphoreType.DMA((N_CHUNKS,)),
        ],
    )(pi, x)


# device time: 186416 ns/iter; 2.1247x vs baseline; 1.2411x over previous
import jax
import jax.numpy as jnp
from jax import lax
from jax.experimental import pallas as pl
from jax.experimental.pallas import tpu as pltpu

C = 8
H = C // 2


def kernel(x, pi):
    _, m, n = x.shape
    quarter = m // 4
    ch = quarter // C

    def body(pi_ref, x_ref, out_ref,
             z_send, z_recv,
             xq_send, xq_recv, yq_send, yq_recv,
             xd_send, xd_recv, yd_send, yd_recv):
        my_x = lax.axis_index("x")
        my_y = lax.axis_index("y")
        my_z = lax.axis_index("z")

        dst_z = pi_ref[my_z]
        nbr_x = 1 - my_x
        ypar = my_y % 2
        nbr_y = my_y + 1 - 2 * ypar

        q_me = 2 * my_x + ypar
        q_xn = 2 * nbr_x + ypar
        q_yp = 2 * my_x + (1 - ypar)
        off_me = q_me * quarter
        off_xn = q_xn * quarter
        off_yp = q_yp * quarter

        z_rdmas = []
        for c in range(C):
            o = off_me + c * ch
            r = pltpu.make_async_remote_copy(
                src_ref=x_ref.at[0, pl.ds(o, ch)],
                dst_ref=out_ref.at[0, pl.ds(o, ch)],
                send_sem=z_send.at[c],
                recv_sem=z_recv.at[c],
                device_id=(my_x, my_y, dst_z),
                device_id_type=pl.DeviceIdType.MESH,
            )
            r.start()
            z_rdmas.append(r)

        xq_rdmas, yq_rdmas = [], []
        for c in range(C):
            z_rdmas[c].wait_recv()
            o = off_me + c * ch
            rx = pltpu.make_async_remote_copy(
                src_ref=out_ref.at[0, pl.ds(o, ch)],
                dst_ref=out_ref.at[0, pl.ds(o, ch)],
                send_sem=xq_send.at[c],
                recv_sem=xq_recv.at[c],
                device_id=(nbr_x, my_y, my_z),
                device_id_type=pl.DeviceIdType.MESH,
            )
            rx.start()
            xq_rdmas.append(rx)
            ry = pltpu.make_async_remote_copy(
                src_ref=out_ref.at[0, pl.ds(o, ch)],
                dst_ref=out_ref.at[0, pl.ds(o, ch)],
                send_sem=yq_send.at[c],
                recv_sem=yq_recv.at[c],
                device_id=(my_x, nbr_y, my_z),
                device_id_type=pl.DeviceIdType.MESH,
            )
            ry.start()
            yq_rdmas.append(ry)

        xd_rdmas, yd_rdmas = [], []
        for c in range(C):
            xq_rdmas[c].wait_recv()
            if c >= H:
                o = off_xn + c * ch
                r = pltpu.make_async_remote_copy(
                    src_ref=out_ref.at[0, pl.ds(o, ch)],
                    dst_ref=out_ref.at[0, pl.ds(o, ch)],
                    send_sem=yd_send.at[c - H],
                    recv_sem=yd_recv.at[c - H],
                    device_id=(my_x, nbr_y, my_z),
                    device_id_type=pl.DeviceIdType.MESH,
                )
                r.start()
                yd_rdmas.append(r)
            yq_rdmas[c].wait_recv()
            if c < H:
                o = off_yp + c * ch
                r = pltpu.make_async_remote_copy(
                    src_ref=out_ref.at[0, pl.ds(o, ch)],
                    dst_ref=out_ref.at[0, pl.ds(o, ch)],
                    send_sem=xd_send.at[c],
                    recv_sem=xd_recv.at[c],
                    device_id=(nbr_x, my_y, my_z),
                    device_id_type=pl.DeviceIdType.MESH,
                )
                r.start()
                xd_rdmas.append(r)

        for i in range(H):
            xd_rdmas[i].wait_recv()
            yd_rdmas[i].wait_recv()

        for c in range(C):
            z_rdmas[c].wait_send()
            xq_rdmas[c].wait_send()
            yq_rdmas[c].wait_send()
        for i in range(H):
            xd_rdmas[i].wait_send()
            yd_rdmas[i].wait_send()

    out_shape = jax.ShapeDtypeStruct(x.shape, jnp.float32)
    return pl.pallas_call(
        body,
        out_shape=out_shape,
        in_specs=[
            pl.BlockSpec(memory_space=pltpu.MemorySpace.SMEM),
            pl.BlockSpec(memory_space=pl.ANY),
        ],
        out_specs=pl.BlockSpec(memory_space=pl.ANY),
        scratch_shapes=[
            pltpu.SemaphoreType.DMA((C,)),
            pltpu.SemaphoreType.DMA((C,)),
            pltpu.SemaphoreType.DMA((C,)),
            pltpu.SemaphoreType.DMA((C,)),
            pltpu.SemaphoreType.DMA((C,)),
            pltpu.SemaphoreType.DMA((C,)),
            pltpu.SemaphoreType.DMA((H,)),
            pltpu.SemaphoreType.DMA((H,)),
            pltpu.SemaphoreType.DMA((H,)),
            pltpu.SemaphoreType.DMA((H,)),
        ],
    )(pi, x)


# device time: 183009 ns/iter; 2.1642x vs baseline; 1.0186x over previous
import functools

import jax
import jax.numpy as jnp
from jax import lax
from jax.experimental import pallas as pl
from jax.experimental.pallas import tpu as pltpu

C = 8
H = C // 2


def kernel(x, pi):
    _, m, n = x.shape
    quarter = m // 4
    ch = quarter // C

    def body(pi_ref, x_ref, out_ref,
             z_send, z_recv,
             xq_send, xq_recv, yq_send, yq_recv,
             xd_send, xd_recv, yd_send, yd_recv):
        my_x = lax.axis_index("x")
        my_y = lax.axis_index("y")
        my_z = lax.axis_index("z")

        dst_z = pi_ref[my_z]
        nbr_x = 1 - my_x
        ypar = my_y % 2
        nbr_y = my_y + 1 - 2 * ypar

        q_me = 2 * my_x + ypar
        q_xn = 2 * nbr_x + ypar
        q_yp = 2 * my_x + (1 - ypar)
        off_me = q_me * quarter
        off_xn = q_xn * quarter
        off_yp = q_yp * quarter

        src_z = (
            0 * (pi_ref[0] == my_z)
            + 1 * (pi_ref[1] == my_z)
            + 2 * (pi_ref[2] == my_z)
            + 3 * (pi_ref[3] == my_z)
        )
        peers = [
            (my_x, my_y, dst_z),
            (my_x, my_y, src_z),
            (nbr_x, my_y, my_z),
            (my_x, nbr_y, my_z),
        ]

        barrier_sem = pltpu.get_barrier_semaphore()
        for p in peers:
            pl.semaphore_signal(
                barrier_sem, inc=1,
                device_id=p, device_id_type=pl.DeviceIdType.MESH,
            )
        pl.semaphore_wait(barrier_sem, len(peers))

        z_rdmas = []
        for c in range(C):
            o = off_me + c * ch
            r = pltpu.make_async_remote_copy(
                src_ref=x_ref.at[0, pl.ds(o, ch)],
                dst_ref=out_ref.at[0, pl.ds(o, ch)],
                send_sem=z_send.at[c],
                recv_sem=z_recv.at[c],
                device_id=(my_x, my_y, dst_z),
                device_id_type=pl.DeviceIdType.MESH,
            )
            r.start()
            z_rdmas.append(r)

        xq_rdmas, yq_rdmas = [], []
        for c in range(C):
            z_rdmas[c].wait_recv()
            o = off_me + c * ch
            rx = pltpu.make_async_remote_copy(
                src_ref=out_ref.at[0, pl.ds(o, ch)],
                dst_ref=out_ref.at[0, pl.ds(o, ch)],
                send_sem=xq_send.at[c],
                recv_sem=xq_recv.at[c],
                device_id=(nbr_x, my_y, my_z),
                device_id_type=pl.DeviceIdType.MESH,
            )
            rx.start()
            xq_rdmas.append(rx)
            ry = pltpu.make_async_remote_copy(
                src_ref=out_ref.at[0, pl.ds(o, ch)],
                dst_ref=out_ref.at[0, pl.ds(o, ch)],
                send_sem=yq_send.at[c],
                recv_sem=yq_recv.at[c],
                device_id=(my_x, nbr_y, my_z),
                device_id_type=pl.DeviceIdType.MESH,
            )
            ry.start()
            yq_rdmas.append(ry)

        xd_rdmas, yd_rdmas = [], []
        for c in range(C):
            xq_rdmas[c].wait_recv()
            if c >= H:
                o = off_xn + c * ch
                r = pltpu.make_async_remote_copy(
                    src_ref=out_ref.at[0, pl.ds(o, ch)],
                    dst_ref=out_ref.at[0, pl.ds(o, ch)],
                    send_sem=yd_send.at[c - H],
                    recv_sem=yd_recv.at[c - H],
                    device_id=(my_x, nbr_y, my_z),
                    device_id_type=pl.DeviceIdType.MESH,
                )
                r.start()
                yd_rdmas.append(r)
            yq_rdmas[c].wait_recv()
            if c < H:
                o = off_yp + c * ch
                r = pltpu.make_async_remote_copy(
                    src_ref=out_ref.at[0, pl.ds(o, ch)],
                    dst_ref=out_ref.at[0, pl.ds(o, ch)],
                    send_sem=xd_send.at[c],
                    recv_sem=xd_recv.at[c],
                    device_id=(nbr_x, my_y, my_z),
                    device_id_type=pl.DeviceIdType.MESH,
                )
                r.start()
                xd_rdmas.append(r)

        for i in range(H):
            xd_rdmas[i].wait_recv()
            yd_rdmas[i].wait_recv()

        for c in range(C):
            z_rdmas[c].wait_send()
            xq_rdmas[c].wait_send()
            yq_rdmas[c].wait_send()
        for i in range(H):
            xd_rdmas[i].wait_send()
            yd_rdmas[i].wait_send()

        @functools.partial(
            pl.run_scoped, second_barrier=pltpu.SemaphoreType.REGULAR
        )
        def _(second_barrier):
            for p in peers:
                pl.semaphore_signal(
                    second_barrier, inc=1,
                    device_id=p, device_id_type=pl.DeviceIdType.MESH,
                )
            pl.semaphore_wait(second_barrier, len(peers))

    out_shape = jax.ShapeDtypeStruct(x.shape, jnp.float32)
    return pl.pallas_call(
        body,
        out_shape=out_shape,
        in_specs=[
            pl.BlockSpec(memory_space=pltpu.MemorySpace.SMEM),
            pl.BlockSpec(memory_space=pl.ANY),
        ],
        out_specs=pl.BlockSpec(memory_space=pl.ANY),
        scratch_shapes=[
            pltpu.SemaphoreType.DMA((C,)),
            pltpu.SemaphoreType.DMA((C,)),
            pltpu.SemaphoreType.DMA((C,)),
            pltpu.SemaphoreType.DMA((C,)),
            pltpu.SemaphoreType.DMA((C,)),
            pltpu.SemaphoreType.DMA((C,)),
            pltpu.SemaphoreType.DMA((H,)),
            pltpu.SemaphoreType.DMA((H,)),
            pltpu.SemaphoreType.DMA((H,)),
            pltpu.SemaphoreType.DMA((H,)),
        ],
        compiler_params=pltpu.CompilerParams(collective_id=0),
    )(pi, x)


# device time: 168700 ns/iter; 2.3478x vs baseline; 1.0848x over previous
import functools

import jax
import jax.numpy as jnp
from jax import lax
from jax.experimental import pallas as pl
from jax.experimental.pallas import tpu as pltpu

C = 8
PZ = 2
XF = range(PZ, PZ + 3)
YF = range(PZ + 3, C)
NF = 3


def kernel(x, pi):
    _, m, n = x.shape
    quarter = m // 4
    ch = quarter // C

    def body(pi_ref, x_ref, out_ref,
             z_send, z_recv,
             xq_send, xq_recv, yq_send, yq_recv,
             xd_send, xd_recv, yd_send, yd_recv):
        my_x = lax.axis_index("x")
        my_y = lax.axis_index("y")
        my_z = lax.axis_index("z")

        dst_z = pi_ref[my_z]
        nbr_x = 1 - my_x
        ypar = my_y % 2
        nbr_y = my_y + 1 - 2 * ypar

        q_me = 2 * my_x + ypar
        q_xn = 2 * nbr_x + ypar
        q_yp = 2 * my_x + (1 - ypar)
        q_dg = 2 * nbr_x + (1 - ypar)
        off_me = q_me * quarter
        off_xn = q_xn * quarter
        off_yp = q_yp * quarter
        off_dg = q_dg * quarter

        src_z = (
            0 * (pi_ref[0] == my_z)
            + 1 * (pi_ref[1] == my_z)
            + 2 * (pi_ref[2] == my_z)
            + 3 * (pi_ref[3] == my_z)
        )
        peers = [
            (my_x, my_y, dst_z),
            (my_x, my_y, src_z),
            (nbr_x, my_y, my_z),
            (my_x, nbr_y, my_z),
        ]

        barrier_sem = pltpu.get_barrier_semaphore()
        for p in peers:
            pl.semaphore_signal(
                barrier_sem, inc=1,
                device_id=p, device_id_type=pl.DeviceIdType.MESH,
            )
        pl.semaphore_wait(barrier_sem, len(peers))

        z_rdmas = []
        for i in range(C + PZ):
            o = (off_me + i * ch) if i < C else (off_dg + (i - C) * ch)
            r = pltpu.make_async_remote_copy(
                src_ref=x_ref.at[0, pl.ds(o, ch)],
                dst_ref=out_ref.at[0, pl.ds(o, ch)],
                send_sem=z_send.at[i],
                recv_sem=z_recv.at[i],
                device_id=(my_x, my_y, dst_z),
                device_id_type=pl.DeviceIdType.MESH,
            )
            r.start()
            z_rdmas.append(r)

        xq_rdmas, yq_rdmas = [], []
        for c in range(C):
            z_rdmas[c].wait_recv()
            o = off_me + c * ch
            rx = pltpu.make_async_remote_copy(
                src_ref=out_ref.at[0, pl.ds(o, ch)],
                dst_ref=out_ref.at[0, pl.ds(o, ch)],
                send_sem=xq_send.at[c],
                recv_sem=xq_recv.at[c],
                device_id=(nbr_x, my_y, my_z),
                device_id_type=pl.DeviceIdType.MESH,
            )
            rx.start()
            xq_rdmas.append(rx)
            ry = pltpu.make_async_remote_copy(
                src_ref=out_ref.at[0, pl.ds(o, ch)],
                dst_ref=out_ref.at[0, pl.ds(o, ch)],
                send_sem=yq_send.at[c],
                recv_sem=yq_recv.at[c],
                device_id=(my_x, nbr_y, my_z),
                device_id_type=pl.DeviceIdType.MESH,
            )
            ry.start()
            yq_rdmas.append(ry)

        xd_rdmas, yd_rdmas = [], []
        for c in range(C):
            yq_rdmas[c].wait_recv()
            if c in XF:
                o = off_yp + c * ch
                r = pltpu.make_async_remote_copy(
                    src_ref=out_ref.at[0, pl.ds(o, ch)],
                    dst_ref=out_ref.at[0, pl.ds(o, ch)],
                    send_sem=xd_send.at[c - PZ],
                    recv_sem=xd_recv.at[c - PZ],
                    device_id=(nbr_x, my_y, my_z),
                    device_id_type=pl.DeviceIdType.MESH,
                )
                r.start()
                xd_rdmas.append(r)
            xq_rdmas[c].wait_recv()
            if c in YF:
                o = off_xn + c * ch
                r = pltpu.make_async_remote_copy(
                    src_ref=out_ref.at[0, pl.ds(o, ch)],
                    dst_ref=out_ref.at[0, pl.ds(o, ch)],
                    send_sem=yd_send.at[c - PZ - NF],
                    recv_sem=yd_recv.at[c - PZ - NF],
                    device_id=(my_x, nbr_y, my_z),
                    device_id_type=pl.DeviceIdType.MESH,
                )
                r.start()
                yd_rdmas.append(r)

        for i in range(PZ):
            z_rdmas[C + i].wait_recv()
        for i in range(NF):
            xd_rdmas[i].wait_recv()
            yd_rdmas[i].wait_recv()

        for i in range(C + PZ):
            z_rdmas[i].wait_send()
        for c in range(C):
            xq_rdmas[c].wait_send()
            yq_rdmas[c].wait_send()
        for i in range(NF):
            xd_rdmas[i].wait_send()
            yd_rdmas[i].wait_send()

        @functools.partial(
            pl.run_scoped, second_barrier=pltpu.SemaphoreType.REGULAR
        )
        def _(second_barrier):
            for p in peers:
                pl.semaphore_signal(
                    second_barrier, inc=1,
                    device_id=p, device_id_type=pl.DeviceIdType.MESH,
                )
            pl.semaphore_wait(second_barrier, len(peers))

    out_shape = jax.ShapeDtypeStruct(x.shape, jnp.float32)
    return pl.pallas_call(
        body,
        out_shape=out_shape,
        in_specs=[
            pl.BlockSpec(memory_space=pltpu.MemorySpace.SMEM),
            pl.BlockSpec(memory_space=pl.ANY),
        ],
        out_specs=pl.BlockSpec(memory_space=pl.ANY),
        scratch_shapes=[
            pltpu.SemaphoreType.DMA((C + PZ,)),
            pltpu.SemaphoreType.DMA((C + PZ,)),
            pltpu.SemaphoreType.DMA((C,)),
            pltpu.SemaphoreType.DMA((C,)),
            pltpu.SemaphoreType.DMA((C,)),
            pltpu.SemaphoreType.DMA((C,)),
            pltpu.SemaphoreType.DMA((NF,)),
            pltpu.SemaphoreType.DMA((NF,)),
            pltpu.SemaphoreType.DMA((NF,)),
            pltpu.SemaphoreType.DMA((NF,)),
        ],
        compiler_params=pltpu.CompilerParams(collective_id=0),
    )(pi, x)


# device time: 166829 ns/iter; 2.3741x vs baseline; 1.0112x over previous
import functools

import jax
import jax.numpy as jnp
from jax import lax
from jax.experimental import pallas as pl
from jax.experimental.pallas import tpu as pltpu

C = 16
PZ = 4
NF = (C - PZ) // 2
XF = range(PZ, PZ + NF)
YF = range(PZ + NF, C)


def kernel(x, pi):
    _, m, n = x.shape
    quarter = m // 4
    ch = quarter // C

    def body(pi_ref, x_ref, out_ref,
             z_send, z_recv,
             xq_send, xq_recv, yq_send, yq_recv,
             xd_send, xd_recv, yd_send, yd_recv):
        my_x = lax.axis_index("x")
        my_y = lax.axis_index("y")
        my_z = lax.axis_index("z")

        dst_z = pi_ref[my_z]
        nbr_x = 1 - my_x
        ypar = my_y % 2
        nbr_y = my_y + 1 - 2 * ypar

        q_me = 2 * my_x + ypar
        q_xn = 2 * nbr_x + ypar
        q_yp = 2 * my_x + (1 - ypar)
        q_dg = 2 * nbr_x + (1 - ypar)
        off_me = q_me * quarter
        off_xn = q_xn * quarter
        off_yp = q_yp * quarter
        off_dg = q_dg * quarter

        src_z = (
            0 * (pi_ref[0] == my_z)
            + 1 * (pi_ref[1] == my_z)
            + 2 * (pi_ref[2] == my_z)
            + 3 * (pi_ref[3] == my_z)
        )
        peers = [
            (my_x, my_y, dst_z),
            (my_x, my_y, src_z),
            (nbr_x, my_y, my_z),
            (my_x, nbr_y, my_z),
        ]

        barrier_sem = pltpu.get_barrier_semaphore()
        for p in peers:
            pl.semaphore_signal(
                barrier_sem, inc=1,
                device_id=p, device_id_type=pl.DeviceIdType.MESH,
            )
        pl.semaphore_wait(barrier_sem, len(peers))

        z_rdmas = []
        for i in range(C + PZ):
            o = (off_me + i * ch) if i < C else (off_dg + (i - C) * ch)
            r = pltpu.make_async_remote_copy(
                src_ref=x_ref.at[0, pl.ds(o, ch)],
                dst_ref=out_ref.at[0, pl.ds(o, ch)],
                send_sem=z_send.at[i],
                recv_sem=z_recv.at[i],
                device_id=(my_x, my_y, dst_z),
                device_id_type=pl.DeviceIdType.MESH,
            )
            r.start()
            z_rdmas.append(r)

        xq_rdmas, yq_rdmas = [], []
        for c in range(C):
            z_rdmas[c].wait_recv()
            o = off_me + c * ch
            rx = pltpu.make_async_remote_copy(
                src_ref=out_ref.at[0, pl.ds(o, ch)],
                dst_ref=out_ref.at[0, pl.ds(o, ch)],
                send_sem=xq_send.at[c],
                recv_sem=xq_recv.at[c],
                device_id=(nbr_x, my_y, my_z),
                device_id_type=pl.DeviceIdType.MESH,
            )
            rx.start()
            xq_rdmas.append(rx)
            ry = pltpu.make_async_remote_copy(
                src_ref=out_ref.at[0, pl.ds(o, ch)],
                dst_ref=out_ref.at[0, pl.ds(o, ch)],
                send_sem=yq_send.at[c],
                recv_sem=yq_recv.at[c],
                device_id=(my_x, nbr_y, my_z),
                device_id_type=pl.DeviceIdType.MESH,
            )
            ry.start()
            yq_rdmas.append(ry)

        xd_rdmas, yd_rdmas = [], []
        for c in range(C):
            yq_rdmas[c].wait_recv()
            if c in XF:
                o = off_yp + c * ch
                r = pltpu.make_async_remote_copy(
                    src_ref=out_ref.at[0, pl.ds(o, ch)],
                    dst_ref=out_ref.at[0, pl.ds(o, ch)],
                    send_sem=xd_send.at[c - PZ],
                    recv_sem=xd_recv.at[c - PZ],
                    device_id=(nbr_x, my_y, my_z),
                    device_id_type=pl.DeviceIdType.MESH,
                )
                r.start()
                xd_rdmas.append(r)
            xq_rdmas[c].wait_recv()
            if c in YF:
                o = off_xn + c * ch
                r = pltpu.make_async_remote_copy(
                    src_ref=out_ref.at[0, pl.ds(o, ch)],
                    dst_ref=out_ref.at[0, pl.ds(o, ch)],
                    send_sem=yd_send.at[c - PZ - NF],
                    recv_sem=yd_recv.at[c - PZ - NF],
                    device_id=(my_x, nbr_y, my_z),
                    device_id_type=pl.DeviceIdType.MESH,
                )
                r.start()
                yd_rdmas.append(r)

        for i in range(PZ):
            z_rdmas[C + i].wait_recv()
        for i in range(NF):
            xd_rdmas[i].wait_recv()
            yd_rdmas[i].wait_recv()

        for i in range(C + PZ):
            z_rdmas[i].wait_send()
        for c in range(C):
            xq_rdmas[c].wait_send()
            yq_rdmas[c].wait_send()
        for i in range(NF):
            xd_rdmas[i].wait_send()
            yd_rdmas[i].wait_send()

        @functools.partial(
            pl.run_scoped, second_barrier=pltpu.SemaphoreType.REGULAR
        )
        def _(second_barrier):
            for p in peers:
                pl.semaphore_signal(
                    second_barrier, inc=1,
                    device_id=p, device_id_type=pl.DeviceIdType.MESH,
                )
            pl.semaphore_wait(second_barrier, len(peers))

    out_shape = jax.ShapeDtypeStruct(x.shape, jnp.float32)
    return pl.pallas_call(
        body,
        out_shape=out_shape,
        in_specs=[
            pl.BlockSpec(memory_space=pltpu.MemorySpace.SMEM),
            pl.BlockSpec(memory_space=pl.ANY),
        ],
        out_specs=pl.BlockSpec(memory_space=pl.ANY),
        scratch_shapes=[
            pltpu.SemaphoreType.DMA((C + PZ,)),
            pltpu.SemaphoreType.DMA((C + PZ,)),
            pltpu.SemaphoreType.DMA((C,)),
            pltpu.SemaphoreType.DMA((C,)),
            pltpu.SemaphoreType.DMA((C,)),
            pltpu.SemaphoreType.DMA((C,)),
            pltpu.SemaphoreType.DMA((NF,)),
            pltpu.SemaphoreType.DMA((NF,)),
            pltpu.SemaphoreType.DMA((NF,)),
            pltpu.SemaphoreType.DMA((NF,)),
        ],
        compiler_params=pltpu.CompilerParams(collective_id=0),
    )(pi, x)


# device time: 166821 ns/iter; 2.3743x vs baseline; 1.0000x over previous
import functools

import jax
import jax.numpy as jnp
from jax import lax
from jax.experimental import pallas as pl
from jax.experimental.pallas import tpu as pltpu

C = 16
PZ = 4
NF = (C - PZ) // 2
XF = range(0, NF)
YF = range(NF, 2 * NF)


def kernel(x, pi):
    _, m, n = x.shape
    quarter = m // 4
    ch = quarter // C

    def body(pi_ref, x_ref, out_ref,
             z_send, z_recv,
             xq_send, xq_recv, yq_send, yq_recv,
             xd_send, xd_recv, yd_send, yd_recv):
        my_x = lax.axis_index("x")
        my_y = lax.axis_index("y")
        my_z = lax.axis_index("z")

        dst_z = pi_ref[my_z]
        nbr_x = 1 - my_x
        ypar = my_y % 2
        nbr_y = my_y + 1 - 2 * ypar

        q_me = 2 * my_x + ypar
        q_xn = 2 * nbr_x + ypar
        q_yp = 2 * my_x + (1 - ypar)
        q_dg = 2 * nbr_x + (1 - ypar)
        off_me = q_me * quarter
        off_xn = q_xn * quarter
        off_yp = q_yp * quarter
        off_dg = q_dg * quarter

        src_z = (
            0 * (pi_ref[0] == my_z)
            + 1 * (pi_ref[1] == my_z)
            + 2 * (pi_ref[2] == my_z)
            + 3 * (pi_ref[3] == my_z)
        )
        peers = [
            (my_x, my_y, dst_z),
            (my_x, my_y, src_z),
            (nbr_x, my_y, my_z),
            (my_x, nbr_y, my_z),
        ]

        barrier_sem = pltpu.get_barrier_semaphore()
        for p in peers:
            pl.semaphore_signal(
                barrier_sem, inc=1,
                device_id=p, device_id_type=pl.DeviceIdType.MESH,
            )
        pl.semaphore_wait(barrier_sem, len(peers))

        z_rdmas = []
        for i in range(C + PZ):
            o = (off_me + i * ch) if i < C else (off_dg + (2 * NF + i - C) * ch)
            r = pltpu.make_async_remote_copy(
                src_ref=x_ref.at[0, pl.ds(o, ch)],
                dst_ref=out_ref.at[0, pl.ds(o, ch)],
                send_sem=z_send.at[i],
                recv_sem=z_recv.at[i],
                device_id=(my_x, my_y, dst_z),
                device_id_type=pl.DeviceIdType.MESH,
            )
            r.start()
            z_rdmas.append(r)

        xq_rdmas, yq_rdmas = [], []
        for c in range(C):
            z_rdmas[c].wait_recv()
            o = off_me + c * ch
            rx = pltpu.make_async_remote_copy(
                src_ref=out_ref.at[0, pl.ds(o, ch)],
                dst_ref=out_ref.at[0, pl.ds(o, ch)],
                send_sem=xq_send.at[c],
                recv_sem=xq_recv.at[c],
                device_id=(nbr_x, my_y, my_z),
                device_id_type=pl.DeviceIdType.MESH,
            )
            rx.start()
            xq_rdmas.append(rx)
            ry = pltpu.make_async_remote_copy(
                src_ref=out_ref.at[0, pl.ds(o, ch)],
                dst_ref=out_ref.at[0, pl.ds(o, ch)],
                send_sem=yq_send.at[c],
                recv_sem=yq_recv.at[c],
                device_id=(my_x, nbr_y, my_z),
                device_id_type=pl.DeviceIdType.MESH,
            )
            ry.start()
            yq_rdmas.append(ry)

        xd_rdmas, yd_rdmas = [], []
        for c in range(C):
            yq_rdmas[c].wait_recv()
            if c in XF:
                o = off_yp + c * ch
                r = pltpu.make_async_remote_copy(
                    src_ref=out_ref.at[0, pl.ds(o, ch)],
                    dst_ref=out_ref.at[0, pl.ds(o, ch)],
                    send_sem=xd_send.at[c],
                    recv_sem=xd_recv.at[c],
                    device_id=(nbr_x, my_y, my_z),
                    device_id_type=pl.DeviceIdType.MESH,
                )
                r.start()
                xd_rdmas.append(r)
            xq_rdmas[c].wait_recv()
            if c in YF:
                o = off_xn + c * ch
                r = pltpu.make_async_remote_copy(
                    src_ref=out_ref.at[0, pl.ds(o, ch)],
                    dst_ref=out_ref.at[0, pl.ds(o, ch)],
                    send_sem=yd_send.at[c - NF],
                    recv_sem=yd_recv.at[c - NF],
                    device_id=(my_x, nbr_y, my_z),
                    device_id_type=pl.DeviceIdType.MESH,
                )
                r.start()
                yd_rdmas.append(r)

        for i in range(PZ):
            z_rdmas[C + i].wait_recv()
        for i in range(NF):
            xd_rdmas[i].wait_recv()
            yd_rdmas[i].wait_recv()

        for i in range(C + PZ):
            z_rdmas[i].wait_send()
        for c in range(C):
            xq_rdmas[c].wait_send()
            yq_rdmas[c].wait_send()
        for i in range(NF):
            xd_rdmas[i].wait_send()
            yd_rdmas[i].wait_send()

        @functools.partial(
            pl.run_scoped, second_barrier=pltpu.SemaphoreType.REGULAR
        )
        def _(second_barrier):
            for p in peers:
                pl.semaphore_signal(
                    second_barrier, inc=1,
                    device_id=p, device_id_type=pl.DeviceIdType.MESH,
                )
            pl.semaphore_wait(second_barrier, len(peers))

    out_shape = jax.ShapeDtypeStruct(x.shape, jnp.float32)
    return pl.pallas_call(
        body,
        out_shape=out_shape,
        in_specs=[
            pl.BlockSpec(memory_space=pltpu.MemorySpace.SMEM),
            pl.BlockSpec(memory_space=pl.ANY),
        ],
        out_specs=pl.BlockSpec(memory_space=pl.ANY),
        scratch_shapes=[
            pltpu.SemaphoreType.DMA((C + PZ,)),
            pltpu.SemaphoreType.DMA((C + PZ,)),
            pltpu.SemaphoreType.DMA((C,)),
            pltpu.SemaphoreType.DMA((C,)),
            pltpu.SemaphoreType.DMA((C,)),
            pltpu.SemaphoreType.DMA((C,)),
            pltpu.SemaphoreType.DMA((NF,)),
            pltpu.SemaphoreType.DMA((NF,)),
            pltpu.SemaphoreType.DMA((NF,)),
            pltpu.SemaphoreType.DMA((NF,)),
        ],
        compiler_params=pltpu.CompilerParams(collective_id=0),
    )(pi, x)
